# Initial kernel scaffold; baseline (speedup 1.0000x reference)
import jax
import jax.numpy as jnp
from jax.experimental import pallas as pl


def kernel(x_sv, x_trk, batch_sv, batch_trk, sv_w1, sv_b1, sv_w2, sv_b2, trk_w1, trk_b1, trk_w2, trk_b2, c1_w, c1_b, c2_w, c2_b, o_w1, o_b1, o_w2, o_b2, o_w3, o_b3, o_w4, o_b4):
    raise NotImplementedError("write your pallas kernel here")


# speedup vs baseline: 3.3460x; 3.3460x over previous
import functools

import jax
import jax.numpy as jnp
from jax import lax
from jax.experimental import pallas as pl
from jax.experimental.pallas import tpu as pltpu
from jax.experimental.pallas import tpu_sc as plsc

HID = 128
K = 8
B = 128
N_SV = 4096
N_TRK = 8192

QB = 256
CT = 256
RB = 512
MASKV = 1.0e30
BIG = 3.0e38
IMAX = 2147483647


def _lin_body(x_ref, w_ref, b_ref, o_ref, *, relu):
    z = jnp.dot(x_ref[...], w_ref[...]) + b_ref[...]
    if relu:
        z = jnp.maximum(z, 0.0)
    o_ref[...] = z


def _lin(x, w, b, relu=False):
    n, kin = x.shape
    return pl.pallas_call(
        functools.partial(_lin_body, relu=relu),
        grid=(n // RB,),
        in_specs=[
            pl.BlockSpec((RB, kin), lambda i: (i, 0)),
            pl.BlockSpec((kin, HID), lambda i: (0, 0)),
            pl.BlockSpec((1, HID), lambda i: (0, 0)),
        ],
        out_specs=pl.BlockSpec((RB, HID), lambda i: (i, 0)),
        out_shape=jax.ShapeDtypeStruct((n, HID), jnp.float32),
    )(x, w, b[None, :])


def _knn_body(tb_ref, q_ref, src_ref, qq_ref, s2_ref, bq_ref, bsrc_ref,
              ss_ref, se_ref, idx_ref, *, n_src):
    i = pl.program_id(0)
    c0 = tb_ref[i, 0]
    c1 = tb_ref[i, 1]
    q = q_ref[...]
    qq = qq_ref[0, 0, :]
    bq = bq_ref[0, 0, :]
    srow = ss_ref[0, 0, :]
    erow = se_ref[0, 0, :]

    seed_rows = []
    for k in range(K):
        r = jnp.where(srow > k, k, erow + (k - srow))
        seed_rows.append(jnp.minimum(r, n_src - 1)[None, :])
    run_i = jnp.concatenate(seed_rows, axis=0)
    run_v = jnp.full((K, QB), MASKV, jnp.float32)

    colio = lax.broadcasted_iota(jnp.int32, (QB, CT), 1)
    rowio = lax.broadcasted_iota(jnp.int32, (2 * K, QB), 0)

    def tile_body(c, carry):
        run_v, run_i = carry
        sv = src_ref[pl.ds(c * CT, CT), :]
        s2 = s2_ref[pl.ds(c, 1), :]
        bs = bsrc_ref[pl.ds(c, 1), :]
        dots = lax.dot_general(q, sv, (((1,), (1,)), ((), ())))
        sc = (qq[:, None] - 2.0 * dots) + s2
        sc = jnp.where(bq[:, None] == bs, sc, MASKV)

        tvs, tis = [], []
        for k in range(K):
            m = jnp.min(sc, axis=1)
            pos = jnp.min(jnp.where(sc == m[:, None], colio, n_src), axis=1)
            tvs.append(m[None, :])
            tis.append((c * CT + pos)[None, :])
            sc = jnp.where(colio == pos[:, None], BIG, sc)
        cat_v = jnp.concatenate([run_v] + tvs, axis=0)
        cat_i = jnp.concatenate([run_i] + tis, axis=0)

        nvs, nis = [], []
        for k in range(K):
            m = jnp.min(cat_v, axis=0)
            pos = jnp.min(jnp.where(cat_v == m[None, :], rowio, 2 * K), axis=0)
            sel = rowio == pos[None, :]
            nvs.append(m[None, :])
            nis.append(jnp.min(jnp.where(sel, cat_i, IMAX), axis=0)[None, :])
            cat_v = jnp.where(sel, BIG, cat_v)
        return (jnp.concatenate(nvs, axis=0), jnp.concatenate(nis, axis=0))

    run_v, run_i = lax.fori_loop(c0, c1, tile_body, (run_v, run_i))
    idx_ref[0] = run_i


def _knn(q, src, qq, s2, bq_r, bsrc_r, ss_r, se_r, tile_bounds):
    n_q = q.shape[0]
    n_src = src.shape[0]
    nblk = n_q // QB
    n_tiles = n_src // CT
    grid_spec = pltpu.PrefetchScalarGridSpec(
        num_scalar_prefetch=1,
        grid=(nblk,),
        in_specs=[
            pl.BlockSpec((QB, HID), lambda i, s: (i, 0)),
            pl.BlockSpec((n_src, HID), lambda i, s: (0, 0)),
            pl.BlockSpec((1, 1, QB), lambda i, s: (i, 0, 0)),
            pl.BlockSpec((n_tiles, CT), lambda i, s: (0, 0)),
            pl.BlockSpec((1, 1, QB), lambda i, s: (i, 0, 0)),
            pl.BlockSpec((n_tiles, CT), lambda i, s: (0, 0)),
            pl.BlockSpec((1, 1, QB), lambda i, s: (i, 0, 0)),
            pl.BlockSpec((1, 1, QB), lambda i, s: (i, 0, 0)),
        ],
        out_specs=pl.BlockSpec((1, K, QB), lambda i, s: (i, 0, 0)),
    )
    return pl.pallas_call(
        functools.partial(_knn_body, n_src=n_src),
        grid_spec=grid_spec,
        out_shape=jax.ShapeDtypeStruct((nblk, K, QB), jnp.int32),
    )(tile_bounds, q, src, qq.reshape(nblk, 1, QB), s2.reshape(n_tiles, CT),
      bq_r, bsrc_r, ss_r, se_r)


_NW = 32
_GCH = 128


def _gather_body(idx_hbm, tab_hbm, out_hbm, idx_v, rows_v, sem):
    w = lax.axis_index("s") * 2 + lax.axis_index("c")
    per_w = (K * N_TRK) // _NW

    def chunk_body(ch, _):
        base = w * per_w + ch * _GCH
        pltpu.sync_copy(idx_hbm.at[pl.ds(base, _GCH)], idx_v)
        pltpu.async_copy(tab_hbm.at[idx_v], rows_v, sem).wait()
        pltpu.sync_copy(rows_v, out_hbm.at[pl.ds(base, _GCH), :])
        return 0

    lax.fori_loop(0, per_w // _GCH, chunk_body, 0)


def _gather_sc(idx_flat, table):
    mesh = plsc.VectorSubcoreMesh(core_axis_name="c", subcore_axis_name="s")
    f = pl.kernel(
        _gather_body,
        out_type=jax.ShapeDtypeStruct((K * N_TRK, HID), jnp.float32),
        mesh=mesh,
        scratch_types=[
            pltpu.VMEM((_GCH,), jnp.int32),
            pltpu.VMEM((_GCH, HID), jnp.float32),
            pltpu.SemaphoreType.DMA,
        ],
    )
    return f(idx_flat, table)


def _edge_body(xi_ref, xjg_ref, w_ref, b_ref, o_ref):
    xi = xi_ref[...]
    acc = None
    for j in range(K):
        xj = xjg_ref[j]
        m = jnp.concatenate([xi, xj - xi], axis=1)
        z = jnp.dot(m, w_ref[...]) + b_ref[...]
        e = jnp.where(z > 0, z, jnp.exp(z) - 1.0)
        acc = e if acc is None else jnp.maximum(acc, e)
    o_ref[...] = acc


def _edge_tc(xi, xjg, w, b):
    nblk = N_TRK // QB
    return pl.pallas_call(
        _edge_body,
        grid=(nblk,),
        in_specs=[
            pl.BlockSpec((QB, HID), lambda i: (i, 0)),
            pl.BlockSpec((K, QB, HID), lambda i: (0, i, 0)),
            pl.BlockSpec((2 * HID, HID), lambda i: (0, 0)),
            pl.BlockSpec((1, HID), lambda i: (0, 0)),
        ],
        out_specs=pl.BlockSpec((QB, HID), lambda i: (i, 0)),
        out_shape=jax.ShapeDtypeStruct((N_TRK, HID), jnp.float32),
    )(xi, xjg, w, b[None, :])


def _pool_body(f_ref, bt_ref, w1_ref, b1_ref, w2_ref, b2_ref, w3_ref, b3_ref,
               w4_ref, b4_ref, out_ref, acc_s, acc_c, *, nblk):
    i = pl.program_id(0)

    @pl.when(i == 0)
    def _():
        acc_s[...] = jnp.zeros_like(acc_s)
        acc_c[...] = jnp.zeros_like(acc_c)

    f = f_ref[...]
    bt = bt_ref[0, 0, :]
    bio = lax.broadcasted_iota(jnp.int32, (B, RB), 0)
    oh = (bio == bt[None, :]).astype(jnp.float32)
    acc_s[...] += lax.dot_general(oh, f, (((1,), (0,)), ((), ())),
                                  precision=lax.Precision.HIGHEST)
    acc_c[...] += jnp.broadcast_to(jnp.sum(oh, axis=1)[:, None], (B, HID))

    @pl.when(i == nblk - 1)
    def _():
        pooled = acc_s[...] / jnp.maximum(acc_c[...], 1.0)

        def layer(h, w_ref, b_ref):
            z = jnp.dot(h, w_ref[...]) + b_ref[...]
            return jnp.where(z > 0, z, jnp.exp(z) - 1.0)

        h = layer(pooled, w1_ref, b1_ref)
        h = layer(h, w2_ref, b2_ref)
        h = layer(h, w3_ref, b3_ref)
        out_ref[...] = jnp.dot(h, w4_ref[...]) + b4_ref[...]


def _pool_head(feats, bt_r, ws):
    nblk = N_TRK // RB
    wspecs = []
    for _ in range(4):
        wspecs += [pl.BlockSpec((HID, HID), lambda i: (0, 0)),
                   pl.BlockSpec((1, HID), lambda i: (0, 0))]
    return pl.pallas_call(
        functools.partial(_pool_body, nblk=nblk),
        grid=(nblk,),
        in_specs=[pl.BlockSpec((RB, HID), lambda i: (i, 0)),
                  pl.BlockSpec((1, 1, RB), lambda i: (i, 0, 0))] + wspecs,
        out_specs=pl.BlockSpec((B, HID), lambda i: (0, 0)),
        out_shape=jax.ShapeDtypeStruct((B, HID), jnp.float32),
        scratch_shapes=[pltpu.VMEM((B, HID), jnp.float32),
                        pltpu.VMEM((B, HID), jnp.float32)],
    )(feats, bt_r, *ws)


def _pad_head_w(w, b):
    wp = jnp.zeros((HID, HID), jnp.float32).at[: w.shape[0], : w.shape[1]].set(w)
    bp = jnp.zeros((1, HID), jnp.float32).at[0, : b.shape[0]].set(b)
    return wp, bp


def kernel(x_sv, x_trk, batch_sv, batch_trk, sv_w1, sv_b1, sv_w2, sv_b2,
           trk_w1, trk_b1, trk_w2, trk_b2, c1_w, c1_b, c2_w, c2_b,
           o_w1, o_b1, o_w2, o_b2, o_w3, o_b3, o_w4, o_b4):
    i32 = jnp.int32

    x_sv_p = jnp.pad(x_sv, ((0, 0), (0, 8 - x_sv.shape[1])))
    sv_w1_p = jnp.pad(sv_w1, ((0, 8 - sv_w1.shape[0]), (0, 0)))
    enc_sv = _lin(jax.nn.elu(_lin(x_sv_p, sv_w1_p, sv_b1)), sv_w2, sv_b2, relu=True)
    enc_trk = _lin(jax.nn.elu(_lin(x_trk, trk_w1, trk_b1)), trk_w2, trk_b2, relu=True)

    qq = jnp.sum(enc_trk * enc_trk, axis=1)
    s2sv = jnp.sum(enc_sv * enc_sv, axis=1)

    nblk = N_TRK // QB
    btr = batch_trk.reshape(nblk, QB)
    bq_lo, bq_hi = btr[:, 0], btr[:, -1]
    bq_r = batch_trk.reshape(nblk, 1, QB)

    ss1 = jnp.searchsorted(batch_sv, batch_trk, side="left").astype(i32)
    se1 = jnp.searchsorted(batch_sv, batch_trk, side="right").astype(i32)
    c0_1 = (jnp.searchsorted(batch_sv, bq_lo, side="left") // CT).astype(i32)
    c1_1 = (-(-jnp.searchsorted(batch_sv, bq_hi, side="right") // CT)).astype(i32)
    tb1 = jnp.stack([c0_1, c1_1], axis=1)

    ss2 = jnp.searchsorted(batch_trk, batch_trk, side="left").astype(i32)
    se2 = jnp.searchsorted(batch_trk, batch_trk, side="right").astype(i32)
    c0_2 = (jnp.searchsorted(batch_trk, bq_lo, side="left") // CT).astype(i32)
    c1_2 = (-(-jnp.searchsorted(batch_trk, bq_hi, side="right") // CT)).astype(i32)
    tb2 = jnp.stack([c0_2, c1_2], axis=1)

    bsv_r = batch_sv.reshape(N_SV // CT, CT)
    btk_r = batch_trk.reshape(N_TRK // CT, CT)

    idx1 = _knn(enc_trk, enc_sv, qq, s2sv, bq_r, bsv_r,
                ss1.reshape(nblk, 1, QB), se1.reshape(nblk, 1, QB), tb1)
    xjg1 = _gather_sc(idx1.transpose(1, 0, 2).reshape(-1), enc_sv)
    feats_1 = _edge_tc(enc_trk, xjg1.reshape(K, N_TRK, HID), c1_w, c1_b)

    s2f = jnp.sum(feats_1 * feats_1, axis=1)
    idx2 = _knn(enc_trk, feats_1, qq, s2f, bq_r, btk_r,
                ss2.reshape(nblk, 1, QB), se2.reshape(nblk, 1, QB), tb2)
    xjg2 = _gather_sc(idx2.transpose(1, 0, 2).reshape(-1), feats_1)
    feats_2 = _edge_tc(enc_trk, xjg2.reshape(K, N_TRK, HID), c2_w, c2_b)

    ws = []
    for w, b in ((o_w1, o_b1), (o_w2, o_b2), (o_w3, o_b3), (o_w4, o_b4)):
        ws += list(_pad_head_w(w, b))
    head = _pool_head(feats_2, batch_trk.reshape(N_TRK // RB, 1, RB), ws)
    out = head[:, :1]
    return (out, jnp.arange(B, dtype=batch_trk.dtype))


# speedup vs baseline: 13.9350x; 4.1646x over previous
import functools

import jax
import jax.numpy as jnp
from jax import lax
from jax.experimental import pallas as pl
from jax.experimental.pallas import tpu as pltpu
from jax.experimental.pallas import tpu_sc as plsc

HID = 128
K = 8
B = 128
N_SV = 4096
N_TRK = 8192

QB = 256
CT = 256
RB = 512
MASKV = 1.0e30
BIG = 3.0e38
IMAX = 2147483647


def _lin_body(x_ref, w_ref, b_ref, o_ref, *, relu):
    z = jnp.dot(x_ref[...], w_ref[0]) + b_ref[0]
    if relu:
        z = jnp.maximum(z, 0.0)
    o_ref[...] = z


def _lin2(x, w_stack, b_stack, split, relu=False):
    n, kin = x.shape

    def wmap(i):
        return ((i >= split).astype(jnp.int32), 0, 0)

    return pl.pallas_call(
        functools.partial(_lin_body, relu=relu),
        grid=(n // RB,),
        in_specs=[
            pl.BlockSpec((RB, kin), lambda i: (i, 0)),
            pl.BlockSpec((1, kin, HID), wmap),
            pl.BlockSpec((1, 1, HID), wmap),
        ],
        out_specs=pl.BlockSpec((RB, HID), lambda i: (i, 0)),
        out_shape=jax.ShapeDtypeStruct((n, HID), jnp.float32),
    )(x, w_stack, b_stack)


def _knn_body(tb_ref, q_ref, src_ref, qq_ref, s2_ref, bq_ref, bsrc_ref,
              off_ref, idx_ref, *, n_src):
    i = pl.program_id(0)
    c0 = tb_ref[i, 0]
    c1 = tb_ref[i, 1]
    q = q_ref[...]
    qq = qq_ref[0, 0, :]
    bq = bq_ref[0, 0, :]

    offs = off_ref[...]
    bio_b = lax.broadcasted_iota(jnp.int32, (QB, B), 1)
    ohq = bq[:, None] == bio_b
    srow = jnp.sum(jnp.where(ohq, offs[0:1, :], 0), axis=1)
    erow = jnp.sum(jnp.where(ohq, offs[1:2, :], 0), axis=1)

    seed_rows = []
    for k in range(K):
        r = jnp.where(srow > k, k, erow + (k - srow))
        seed_rows.append(jnp.minimum(r, n_src - 1)[None, :])
    run_i = jnp.concatenate(seed_rows, axis=0)
    run_v = jnp.full((K, QB), MASKV, jnp.float32)

    colio = lax.broadcasted_iota(jnp.int32, (QB, CT), 1)
    rowio = lax.broadcasted_iota(jnp.int32, (2 * K, QB), 0)

    def tile_body(c, carry):
        run_v, run_i = carry
        sv = src_ref[pl.ds(c * CT, CT), :]
        s2 = s2_ref[pl.ds(c, 1), :]
        bs = bsrc_ref[pl.ds(c, 1), :]
        dots = lax.dot_general(q, sv, (((1,), (1,)), ((), ())))
        sc = (qq[:, None] - 2.0 * dots) + s2
        sc = jnp.where(bq[:, None] == bs, sc, MASKV)

        tvs, tis = [], []
        for k in range(K):
            m = jnp.min(sc, axis=1)
            pos = jnp.min(jnp.where(sc == m[:, None], colio, n_src), axis=1)
            tvs.append(m[None, :])
            tis.append((c * CT + pos)[None, :])
            sc = jnp.where(colio == pos[:, None], BIG, sc)
        cat_v = jnp.concatenate([run_v] + tvs, axis=0)
        cat_i = jnp.concatenate([run_i] + tis, axis=0)

        nvs, nis = [], []
        for k in range(K):
            m = jnp.min(cat_v, axis=0)
            pos = jnp.min(jnp.where(cat_v == m[None, :], rowio, 2 * K), axis=0)
            sel = rowio == pos[None, :]
            nvs.append(m[None, :])
            nis.append(jnp.min(jnp.where(sel, cat_i, IMAX), axis=0)[None, :])
            cat_v = jnp.where(sel, BIG, cat_v)
        return (jnp.concatenate(nvs, axis=0), jnp.concatenate(nis, axis=0))

    run_v, run_i = lax.fori_loop(c0, c1, tile_body, (run_v, run_i))
    idx_ref[0] = run_i


def _knn(q, src, qq, s2, bq_r, bsrc_r, offs, tile_bounds):
    n_q = q.shape[0]
    n_src = src.shape[0]
    nblk = n_q // QB
    n_tiles = n_src // CT
    grid_spec = pltpu.PrefetchScalarGridSpec(
        num_scalar_prefetch=1,
        grid=(nblk,),
        in_specs=[
            pl.BlockSpec((QB, HID), lambda i, s: (i, 0)),
            pl.BlockSpec((n_src, HID), lambda i, s: (0, 0)),
            pl.BlockSpec((1, 1, QB), lambda i, s: (i, 0, 0)),
            pl.BlockSpec((n_tiles, CT), lambda i, s: (0, 0)),
            pl.BlockSpec((1, 1, QB), lambda i, s: (i, 0, 0)),
            pl.BlockSpec((n_tiles, CT), lambda i, s: (0, 0)),
            pl.BlockSpec((2, B), lambda i, s: (0, 0)),
        ],
        out_specs=pl.BlockSpec((1, K, QB), lambda i, s: (i, 0, 0)),
    )
    return pl.pallas_call(
        functools.partial(_knn_body, n_src=n_src),
        grid_spec=grid_spec,
        out_shape=jax.ShapeDtypeStruct((nblk, K, QB), jnp.int32),
    )(tile_bounds, q, src, qq.reshape(nblk, 1, QB), s2.reshape(n_tiles, CT),
      bq_r, bsrc_r, offs)


_NW = 32
_GCH = 128


def _gather_body(idx_hbm, tab_hbm, out_hbm, idx_v, rows_v, sem0, sem1):
    w = lax.axis_index("s") * 2 + lax.axis_index("c")
    per_w = (K * N_TRK) // _NW
    nch = per_w // _GCH
    base = w * per_w
    pltpu.sync_copy(idx_hbm.at[pl.ds(base, per_w)], idx_v)
    sems = (sem0, sem1)

    def copy(ch, b):
        return pltpu.make_async_copy(
            tab_hbm.at[idx_v.at[pl.ds(ch * _GCH, _GCH)]], rows_v.at[b], sems[b])

    copy(0, 0).start()
    copy(1, 1).start()

    def body(i, _):
        for b in range(2):
            ch = i * 2 + b
            copy(ch, b).wait()
            pltpu.sync_copy(rows_v.at[b],
                            out_hbm.at[pl.ds(base + ch * _GCH, _GCH), :])

            @pl.when(ch + 2 < nch)
            def _():
                copy(ch + 2, b).start()
        return 0

    lax.fori_loop(0, nch // 2, body, 0)


def _gather_sc(idx_flat, table):
    mesh = plsc.VectorSubcoreMesh(core_axis_name="c", subcore_axis_name="s")
    f = pl.kernel(
        _gather_body,
        out_type=jax.ShapeDtypeStruct((K * N_TRK, HID), jnp.float32),
        mesh=mesh,
        scratch_types=[
            pltpu.VMEM(((K * N_TRK) // _NW,), jnp.int32),
            pltpu.VMEM((2, _GCH, HID), jnp.float32),
            pltpu.SemaphoreType.DMA,
            pltpu.SemaphoreType.DMA,
        ],
    )
    return f(idx_flat, table)


def _edge_body(xi_ref, xjg_ref, w_ref, b_ref, o_ref):
    xi = xi_ref[...]
    acc = None
    for j in range(K):
        xj = xjg_ref[j]
        m = jnp.concatenate([xi, xj - xi], axis=1)
        z = jnp.dot(m, w_ref[...]) + b_ref[...]
        e = jnp.where(z > 0, z, jnp.exp(z) - 1.0)
        acc = e if acc is None else jnp.maximum(acc, e)
    o_ref[...] = acc


def _edge_tc(xi, xjg, w, b):
    nblk = N_TRK // QB
    return pl.pallas_call(
        _edge_body,
        grid=(nblk,),
        in_specs=[
            pl.BlockSpec((QB, HID), lambda i: (i, 0)),
            pl.BlockSpec((K, QB, HID), lambda i: (0, i, 0)),
            pl.BlockSpec((2 * HID, HID), lambda i: (0, 0)),
            pl.BlockSpec((1, HID), lambda i: (0, 0)),
        ],
        out_specs=pl.BlockSpec((QB, HID), lambda i: (i, 0)),
        out_shape=jax.ShapeDtypeStruct((N_TRK, HID), jnp.float32),
    )(xi, xjg, w, b[None, :])


def _pool_body(f_ref, bt_ref, w1_ref, b1_ref, w2_ref, b2_ref, w3_ref, b3_ref,
               w4_ref, b4_ref, out_ref, acc_s, acc_c, *, nblk):
    i = pl.program_id(0)

    @pl.when(i == 0)
    def _():
        acc_s[...] = jnp.zeros_like(acc_s)
        acc_c[...] = jnp.zeros_like(acc_c)

    f = f_ref[...]
    bt = bt_ref[0, 0, :]
    bio = lax.broadcasted_iota(jnp.int32, (B, RB), 0)
    oh = (bio == bt[None, :]).astype(jnp.float32)
    acc_s[...] += lax.dot_general(oh, f, (((1,), (0,)), ((), ())),
                                  precision=lax.Precision.HIGHEST)
    acc_c[...] += jnp.broadcast_to(jnp.sum(oh, axis=1)[:, None], (B, HID))

    @pl.when(i == nblk - 1)
    def _():
        pooled = acc_s[...] / jnp.maximum(acc_c[...], 1.0)

        def layer(h, w_ref, b_ref):
            z = jnp.dot(h, w_ref[...]) + b_ref[...]
            return jnp.where(z > 0, z, jnp.exp(z) - 1.0)

        h = layer(pooled, w1_ref, b1_ref)
        h = layer(h, w2_ref, b2_ref)
        h = layer(h, w3_ref, b3_ref)
        out_ref[...] = jnp.dot(h, w4_ref[...]) + b4_ref[...]


def _pool_head(feats, bt_r, ws):
    nblk = N_TRK // RB
    wspecs = []
    for _ in range(4):
        wspecs += [pl.BlockSpec((HID, HID), lambda i: (0, 0)),
                   pl.BlockSpec((1, HID), lambda i: (0, 0))]
    return pl.pallas_call(
        functools.partial(_pool_body, nblk=nblk),
        grid=(nblk,),
        in_specs=[pl.BlockSpec((RB, HID), lambda i: (i, 0)),
                  pl.BlockSpec((1, 1, RB), lambda i: (i, 0, 0))] + wspecs,
        out_specs=pl.BlockSpec((B, HID), lambda i: (0, 0)),
        out_shape=jax.ShapeDtypeStruct((B, HID), jnp.float32),
        scratch_shapes=[pltpu.VMEM((B, HID), jnp.float32),
                        pltpu.VMEM((B, HID), jnp.float32)],
    )(feats, bt_r, *ws)


def _pad_head_w(w, b):
    wp = jnp.zeros((HID, HID), jnp.float32).at[: w.shape[0], : w.shape[1]].set(w)
    bp = jnp.zeros((1, HID), jnp.float32).at[0, : b.shape[0]].set(b)
    return wp, bp


def kernel(x_sv, x_trk, batch_sv, batch_trk, sv_w1, sv_b1, sv_w2, sv_b2,
           trk_w1, trk_b1, trk_w2, trk_b2, c1_w, c1_b, c2_w, c2_b,
           o_w1, o_b1, o_w2, o_b2, o_w3, o_b3, o_w4, o_b4):
    i32 = jnp.int32

    x_sv_p = jnp.pad(x_sv, ((0, 0), (0, 8 - x_sv.shape[1])))
    sv_w1_p = jnp.pad(sv_w1, ((0, 8 - sv_w1.shape[0]), (0, 0)))
    x_all = jnp.concatenate([x_sv_p, x_trk], axis=0)
    w1s = jnp.stack([sv_w1_p, trk_w1])
    b1s = jnp.stack([sv_b1[None, :], trk_b1[None, :]])
    w2s = jnp.stack([sv_w2, trk_w2])
    b2s = jnp.stack([sv_b2[None, :], trk_b2[None, :]])
    split = N_SV // RB
    z_all = _lin2(x_all, w1s, b1s, split)
    enc_all = _lin2(jax.nn.elu(z_all), w2s, b2s, split, relu=True)
    enc_sv, enc_trk = enc_all[:N_SV], enc_all[N_SV:]

    qq = jnp.sum(enc_trk * enc_trk, axis=1)
    s2sv = jnp.sum(enc_sv * enc_sv, axis=1)

    nblk = N_TRK // QB
    btr = batch_trk.reshape(nblk, QB)
    bq_lo, bq_hi = btr[:, 0:1], btr[:, -1:]
    bq_r = batch_trk.reshape(nblk, 1, QB)
    bio = jnp.arange(B, dtype=i32)[None, :]

    def _bounds(batch_src):
        hist = jnp.sum(batch_src[None, :] == jnp.arange(B, dtype=i32)[:, None],
                       axis=1).astype(i32)
        off_e = jnp.cumsum(hist).astype(i32)
        off_s = off_e - hist
        offs = jnp.stack([off_s, off_e])
        c0 = jnp.sum(jnp.where(bio < bq_lo, hist[None, :], 0), axis=1) // CT
        cnt_le = jnp.sum(jnp.where(bio <= bq_hi, hist[None, :], 0), axis=1)
        c1 = -(-cnt_le // CT)
        return offs, jnp.stack([c0, c1], axis=1).astype(i32)

    offs1, tb1 = _bounds(batch_sv)
    offs2, tb2 = _bounds(batch_trk)

    bsv_r = batch_sv.reshape(N_SV // CT, CT)
    btk_r = batch_trk.reshape(N_TRK // CT, CT)

    idx1 = _knn(enc_trk, enc_sv, qq, s2sv, bq_r, bsv_r, offs1, tb1)
    xjg1 = _gather_sc(idx1.transpose(1, 0, 2).reshape(-1), enc_sv)
    feats_1 = _edge_tc(enc_trk, xjg1.reshape(K, N_TRK, HID), c1_w, c1_b)

    s2f = jnp.sum(feats_1 * feats_1, axis=1)
    idx2 = _knn(enc_trk, feats_1, qq, s2f, bq_r, btk_r, offs2, tb2)
    xjg2 = _gather_sc(idx2.transpose(1, 0, 2).reshape(-1), feats_1)
    feats_2 = _edge_tc(enc_trk, xjg2.reshape(K, N_TRK, HID), c2_w, c2_b)

    ws = []
    for w, b in ((o_w1, o_b1), (o_w2, o_b2), (o_w3, o_b3), (o_w4, o_b4)):
        ws += list(_pad_head_w(w, b))
    head = _pool_head(feats_2, batch_trk.reshape(N_TRK // RB, 1, RB), ws)
    out = head[:, :1]
    return (out, jnp.arange(B, dtype=batch_trk.dtype))
